# Initial kernel scaffold; baseline (speedup 1.0000x reference)
import jax
import jax.numpy as jnp
from jax.experimental import pallas as pl


def kernel(tokens, edge_index, eta, emb, W, bias):
    raise NotImplementedError("write your pallas kernel here")


# speedup vs baseline: 14.6314x; 14.6314x over previous
import functools

import jax
import jax.numpy as jnp
from jax import lax
from jax.experimental import pallas as pl
from jax.experimental.pallas import tpu as pltpu
from jax.experimental.pallas import tpu_sc as plsc

N = 10000
E = 160000
V = 1000
F = 256
C = 256
HALF = C // 2
NCORE = 2
NSUB = 16
EPW = E // NSUB
B = 80
NCHUNK = EPW // B
ROWS_PT = N // NSUB
FCH = 25
NF = ROWS_PT // FCH


def _mm_body(emb_ref, wp_ref, out_ref):
    out_ref[0] = jnp.dot(emb_ref[...], wp_ref[0],
                         preferred_element_type=jnp.float32)


def _make_table(emb, Wp):
    return pl.pallas_call(
        _mm_body,
        grid=(NCORE,),
        in_specs=[
            pl.BlockSpec((V, F), lambda s: (0, 0)),
            pl.BlockSpec((1, F, 3 * HALF), lambda s: (s, 0, 0)),
        ],
        out_specs=pl.BlockSpec((1, V, 3 * HALF), lambda s: (s, 0, 0)),
        out_shape=jax.ShapeDtypeStruct((NCORE, V, 3 * HALF), jnp.float32),
    )(emb, Wp)


_sc_mesh = plsc.VectorSubcoreMesh(core_axis_name="c", subcore_axis_name="s")


@functools.partial(
    pl.kernel,
    out_type=jax.ShapeDtypeStruct((NCORE * NSUB, 16), jnp.float32),
    mesh=_sc_mesh,
    scratch_types=[
        pltpu.VMEM((B,), jnp.int32),
        pltpu.VMEM((B,), jnp.int32),
        pltpu.VMEM((B,), jnp.int32),
        pltpu.VMEM((B * 4 + 16, ), jnp.float32),
        pltpu.VMEM((B, 3 * HALF), jnp.float32),
        pltpu.VMEM((B, HALF), jnp.float32),
        pltpu.VMEM((FCH, HALF), jnp.float32),
        pltpu.VMEM((HALF,), jnp.float32),
        pltpu.VMEM((16,), jnp.float32),
        pltpu.VMEM_SHARED((N, HALF), jnp.float32),
        pltpu.SemaphoreType.DMA,
    ],
)
def _sc_edge(src_hbm, dst_hbm, eta_hbm, tok_hbm, t_hbm, bias_hbm, out_hbm,
             src_v, dst_v, tp_v, eta_v, rows_v, msgs_v, fbuf, bias_v,
             accst, agg_sh, sem):
    cid = lax.axis_index("c")
    sid = lax.axis_index("s")
    wid = cid * NSUB + sid

    pltpu.sync_copy(bias_hbm.at[cid], bias_v)

    zero16 = jnp.zeros((16,), jnp.float32)

    def _zrow(r, carry):
        for j in range(HALF // 16):
            fbuf[r, pl.ds(16 * j, 16)] = zero16
        return carry

    lax.fori_loop(0, FCH, _zrow, 0)
    for p in range(NF):
        pltpu.sync_copy(
            fbuf, agg_sh.at[pl.ds(sid * ROWS_PT + p * FCH, FCH), :])
    plsc.subcore_barrier()

    ebase = sid * EPW
    coff = jnp.full((16,), cid * V, jnp.int32)

    def _chunk(i, carry):
        base = ebase + i * B
        pltpu.sync_copy(src_hbm.at[pl.ds(base, B)], src_v)
        pltpu.sync_copy(dst_hbm.at[pl.ds(base, B)], dst_v)
        pltpu.sync_copy(eta_hbm.at[pl.ds(base * 4, B * 4)],
                        eta_v.at[pl.ds(0, B * 4)])
        pltpu.async_copy(tok_hbm.at[src_v], tp_v, sem).wait()
        for j in range(B // 16):
            tp_v[pl.ds(16 * j, 16)] = tp_v[pl.ds(16 * j, 16)] + coff
        pltpu.async_copy(t_hbm.at[tp_v], rows_v, sem).wait()

        def _edge(b, c2):
            ev = eta_v[pl.ds(4 * b, 16)]
            e0 = ev[0]
            e1 = ev[1]
            e2 = ev[2]
            for j in range(HALF // 16):
                r0 = rows_v[b, pl.ds(16 * j, 16)]
                r1 = rows_v[b, pl.ds(HALF + 16 * j, 16)]
                r2 = rows_v[b, pl.ds(2 * HALF + 16 * j, 16)]
                msgs_v[b, pl.ds(16 * j, 16)] = e0 * r0 + e1 * r1 + e2 * r2
            return c2

        lax.fori_loop(0, B, _edge, 0)
        pltpu.sync_copy(msgs_v, agg_sh.at[dst_v], add=True)
        return carry

    lax.fori_loop(0, NCHUNK, _chunk, 0)
    plsc.subcore_barrier()

    def _piece(p, acc):
        pltpu.sync_copy(
            agg_sh.at[pl.ds(sid * ROWS_PT + p * FCH, FCH), :], fbuf)

        def _row(r, acc2):
            out = []
            for j in range(HALF // 16):
                x = fbuf[r, pl.ds(16 * j, 16)] + bias_v[pl.ds(16 * j, 16)]
                ex = jnp.exp(2.0 * x)
                th = 1.0 - 2.0 / (ex + 1.0)
                out.append(acc2[j] + th)
            return tuple(out)

        return lax.fori_loop(0, FCH, _row, acc)

    acc0 = tuple(jnp.zeros((16,), jnp.float32) for _ in range(HALF // 16))
    acc = lax.fori_loop(0, NF, _piece, acc0)
    total = acc[0]
    for j in range(1, HALF // 16):
        total = total + acc[j]
    accst[...] = total * (1.0 / (N * C))
    pltpu.sync_copy(accst, out_hbm.at[wid])


def kernel(tokens, edge_index, eta, emb, W, bias):
    src = edge_index[0]
    dst = edge_index[1]
    eta4 = jnp.pad(eta.reshape(E, 3), ((0, 0), (0, 1))).reshape(E * 4)
    Wp = (W.reshape(F, 3, NCORE, HALF)
          .transpose(2, 0, 1, 3)
          .reshape(NCORE, F, 3 * HALF))
    bias2 = bias.reshape(NCORE, HALF)
    table = _make_table(emb, Wp).reshape(NCORE * V, 3 * HALF)
    partials = _sc_edge(src, dst, eta4, tokens, table, bias2)
    return jnp.sum(partials)


# speedup vs baseline: 21.8706x; 1.4948x over previous
import functools

import jax
import jax.numpy as jnp
from jax import lax
from jax.experimental import pallas as pl
from jax.experimental.pallas import tpu as pltpu
from jax.experimental.pallas import tpu_sc as plsc

N = 10000
E = 160000
V = 1000
F = 256
C = 256
HALF = C // 2
NCORE = 2
NSUB = 16
EPW = E // NSUB
B = 80
NCHUNK = EPW // B
ROWS_PT = N // NSUB
FCH = 25
NF = ROWS_PT // FCH


def _mm_body(emb_ref, wp_ref, out_ref):
    out_ref[0] = jnp.dot(emb_ref[...], wp_ref[0],
                         preferred_element_type=jnp.float32)


def _make_table(emb, Wp):
    return pl.pallas_call(
        _mm_body,
        grid=(NCORE,),
        in_specs=[
            pl.BlockSpec((V, F), lambda s: (0, 0)),
            pl.BlockSpec((1, F, 3 * HALF), lambda s: (s, 0, 0)),
        ],
        out_specs=pl.BlockSpec((1, V, 3 * HALF), lambda s: (s, 0, 0)),
        out_shape=jax.ShapeDtypeStruct((NCORE, V, 3 * HALF), jnp.float32),
    )(emb, Wp)


_sc_mesh = plsc.VectorSubcoreMesh(core_axis_name="c", subcore_axis_name="s")


@functools.partial(
    pl.kernel,
    out_type=jax.ShapeDtypeStruct((NCORE * NSUB, 16), jnp.float32),
    mesh=_sc_mesh,
    scratch_types=[
        pltpu.VMEM((B,), jnp.int32),
        pltpu.VMEM((B,), jnp.int32),
        pltpu.VMEM((B,), jnp.int32),
        pltpu.VMEM((B * 4 + 16, ), jnp.float32),
        pltpu.VMEM((B, 3 * HALF), jnp.float32),
        pltpu.VMEM((B, HALF), jnp.float32),
        pltpu.VMEM((FCH, HALF), jnp.float32),
        pltpu.VMEM((HALF,), jnp.float32),
        pltpu.VMEM((16,), jnp.float32),
        pltpu.VMEM_SHARED((N, HALF), jnp.float32),
        pltpu.SemaphoreType.DMA,
    ],
)
def _sc_edge(src_hbm, dst_hbm, eta_hbm, tok_hbm, t_hbm, bias_hbm, out_hbm,
             src_v, dst_v, tp_v, eta_v, rows_v, msgs_v, fbuf, bias_v,
             accst, agg_sh, sem):
    cid = lax.axis_index("c")
    sid = lax.axis_index("s")
    wid = cid * NSUB + sid

    pltpu.sync_copy(bias_hbm.at[cid], bias_v)

    zero16 = jnp.zeros((16,), jnp.float32)

    def _zrow(r, carry):
        for j in range(HALF // 16):
            fbuf[r, pl.ds(16 * j, 16)] = zero16
        return carry

    lax.fori_loop(0, FCH, _zrow, 0)
    for p in range(NF):
        pltpu.sync_copy(
            fbuf, agg_sh.at[pl.ds(sid * ROWS_PT + p * FCH, FCH), :])
    plsc.subcore_barrier()

    ebase = sid * EPW
    coff = jnp.full((16,), cid * V, jnp.int32)

    def _chunk(i, carry):
        base = ebase + i * B
        pltpu.sync_copy(src_hbm.at[pl.ds(base, B)], src_v)
        pltpu.sync_copy(dst_hbm.at[pl.ds(base, B)], dst_v)
        pltpu.sync_copy(eta_hbm.at[pl.ds(base * 4, B * 4)],
                        eta_v.at[pl.ds(0, B * 4)])
        pltpu.async_copy(tok_hbm.at[src_v], tp_v, sem).wait()
        for j in range(B // 16):
            tp_v[pl.ds(16 * j, 16)] = tp_v[pl.ds(16 * j, 16)] + coff
        pltpu.async_copy(t_hbm.at[tp_v], rows_v, sem).wait()

        @plsc.parallel_loop(0, B, 1, unroll=4)
        def _edge(b):
            ev = eta_v[pl.ds(4 * b, 16)]
            e0 = ev[0]
            e1 = ev[1]
            e2 = ev[2]
            for j in range(HALF // 16):
                r0 = rows_v[b, pl.ds(16 * j, 16)]
                r1 = rows_v[b, pl.ds(HALF + 16 * j, 16)]
                r2 = rows_v[b, pl.ds(2 * HALF + 16 * j, 16)]
                msgs_v[b, pl.ds(16 * j, 16)] = e0 * r0 + e1 * r1 + e2 * r2
        pltpu.sync_copy(msgs_v, agg_sh.at[dst_v], add=True)
        return carry

    lax.fori_loop(0, NCHUNK, _chunk, 0)
    plsc.subcore_barrier()

    def _piece(p, acc):
        pltpu.sync_copy(
            agg_sh.at[pl.ds(sid * ROWS_PT + p * FCH, FCH), :], fbuf)

        @plsc.parallel_loop(0, FCH, 1, unroll=2, carry=acc)
        def _row(r, acc2):
            out = []
            for j in range(HALF // 16):
                x = fbuf[r, pl.ds(16 * j, 16)] + bias_v[pl.ds(16 * j, 16)]
                ex = jnp.exp(2.0 * x)
                th = 1.0 - 2.0 / (ex + 1.0)
                out.append(acc2[j] + th)
            return tuple(out)

        return _row

    acc0 = tuple(jnp.zeros((16,), jnp.float32) for _ in range(HALF // 16))
    acc = lax.fori_loop(0, NF, _piece, acc0)
    total = acc[0]
    for j in range(1, HALF // 16):
        total = total + acc[j]
    accst[...] = total * (1.0 / (N * C))
    pltpu.sync_copy(accst, out_hbm.at[wid])


def kernel(tokens, edge_index, eta, emb, W, bias):
    src = edge_index[0]
    dst = edge_index[1]
    eta4 = jnp.pad(eta.reshape(E, 3), ((0, 0), (0, 1))).reshape(E * 4)
    Wp = (W.reshape(F, 3, NCORE, HALF)
          .transpose(2, 0, 1, 3)
          .reshape(NCORE, F, 3 * HALF))
    bias2 = bias.reshape(NCORE, HALF)
    table = _make_table(emb, Wp).reshape(NCORE * V, 3 * HALF)
    partials = _sc_edge(src, dst, eta4, tokens, table, bias2)
    return jnp.sum(partials)


# speedup vs baseline: 35.7181x; 1.6332x over previous
import functools

import jax
import jax.numpy as jnp
from jax import lax
from jax.experimental import pallas as pl
from jax.experimental.pallas import tpu as pltpu
from jax.experimental.pallas import tpu_sc as plsc

N = 10000
E = 160000
V = 1000
F = 256
C = 256
HALF = C // 2
NCORE = 2
NSUB = 16
EPW = E // NSUB
B = 40
NCHUNK = EPW // B
ROWS_PT = N // NSUB
FCH = 25
NF = ROWS_PT // FCH


def _mm_body(emb_ref, wp_ref, out_ref):
    out_ref[0] = jnp.dot(emb_ref[...], wp_ref[0],
                         preferred_element_type=jnp.float32)


def _make_table(emb, Wp):
    return pl.pallas_call(
        _mm_body,
        grid=(NCORE,),
        in_specs=[
            pl.BlockSpec((V, F), lambda s: (0, 0)),
            pl.BlockSpec((1, F, 3 * HALF), lambda s: (s, 0, 0)),
        ],
        out_specs=pl.BlockSpec((1, V, 3 * HALF), lambda s: (s, 0, 0)),
        out_shape=jax.ShapeDtypeStruct((NCORE, V, 3 * HALF), jnp.float32),
    )(emb, Wp)


_sc_mesh = plsc.VectorSubcoreMesh(core_axis_name="c", subcore_axis_name="s")


@functools.partial(
    pl.kernel,
    out_type=jax.ShapeDtypeStruct((NCORE * NSUB, 16), jnp.float32),
    mesh=_sc_mesh,
    scratch_types=[
        pltpu.VMEM((2, B), jnp.int32),
        pltpu.VMEM((4, B), jnp.int32),
        pltpu.VMEM((2, B), jnp.int32),
        pltpu.VMEM((4 * (B * 4 + 16),), jnp.float32),
        pltpu.VMEM((2, B, 3 * HALF), jnp.float32),
        pltpu.VMEM((B, HALF), jnp.float32),
        pltpu.VMEM((FCH, HALF), jnp.float32),
        pltpu.VMEM((HALF,), jnp.float32),
        pltpu.VMEM((16,), jnp.float32),
        pltpu.VMEM_SHARED((N, HALF), jnp.float32),
        pltpu.SemaphoreType.DMA,
        pltpu.SemaphoreType.DMA,
        pltpu.SemaphoreType.DMA,
    ],
)
def _sc_edge(src_hbm, dst_hbm, eta_hbm, tok_hbm, t_hbm, bias_hbm, out_hbm,
             src_v, dst_v, tp_v, eta_v, rows_v, msgs_v, fbuf, bias_v,
             accst, agg_sh, sem_a, sem_b, sem_c):
    cid = lax.axis_index("c")
    sid = lax.axis_index("s")
    wid = cid * NSUB + sid

    pltpu.sync_copy(bias_hbm.at[cid], bias_v)

    zero16 = jnp.zeros((16,), jnp.float32)

    def _zrow(r, carry):
        for j in range(HALF // 16):
            fbuf[r, pl.ds(16 * j, 16)] = zero16
        return carry

    lax.fori_loop(0, FCH, _zrow, 0)
    for p in range(NF):
        pltpu.sync_copy(
            fbuf, agg_sh.at[pl.ds(sid * ROWS_PT + p * FCH, FCH), :])
    plsc.subcore_barrier()

    ebase = sid * EPW

    def _a_copies(ch):
        base = ebase + ch * B
        return (
            (src_hbm.at[pl.ds(base, B)], src_v.at[ch % 2]),
            (dst_hbm.at[pl.ds(base, B)], dst_v.at[ch % 4]),
            (eta_hbm.at[pl.ds(base * 4, B * 4 + 16)],
             eta_v.at[pl.ds((ch % 4) * (B * 4 + 16), B * 4 + 16)]),
        )

    def _b_copy(ch):
        return (tok_hbm.at[src_v.at[ch % 2]], tp_v.at[ch % 2])

    def _c_copy(ch):
        return (t_hbm.at[cid].at[tp_v.at[ch % 2]], rows_v.at[ch % 2])

    def _pipe(j, carry):
        @pl.when(j >= 3)
        def _():
            pltpu.make_async_copy(*_c_copy(j - 3), sem_c).wait()

        @pl.when(jnp.logical_and(j >= 2, j <= NCHUNK + 1))
        def _():
            pltpu.make_async_copy(*_b_copy(j - 2), sem_b).wait()
            pltpu.async_copy(*_c_copy(j - 2), sem_c)

        @pl.when(jnp.logical_and(j >= 1, j <= NCHUNK))
        def _():
            for s_d in _a_copies(j - 1):
                pltpu.make_async_copy(*s_d, sem_a).wait()
            pltpu.async_copy(*_b_copy(j - 1), sem_b)

        @pl.when(j <= NCHUNK - 1)
        def _():
            for s_d in _a_copies(j):
                pltpu.async_copy(*s_d, sem_a)

        @pl.when(j >= 3)
        def _():
            ch = j - 3
            r2 = ch % 2
            r4 = ch % 4

            @plsc.parallel_loop(0, B, 1, unroll=4)
            def _edge(b):
                ev = eta_v[pl.ds(r4 * (B * 4 + 16) + 4 * b, 16)]
                e0 = ev[0]
                e1 = ev[1]
                e2 = ev[2]
                for j2 in range(HALF // 16):
                    r0 = rows_v[r2, b, pl.ds(16 * j2, 16)]
                    r1 = rows_v[r2, b, pl.ds(HALF + 16 * j2, 16)]
                    r2v = rows_v[r2, b, pl.ds(2 * HALF + 16 * j2, 16)]
                    msgs_v[b, pl.ds(16 * j2, 16)] = (
                        e0 * r0 + e1 * r1 + e2 * r2v)

            pltpu.sync_copy(msgs_v, agg_sh.at[dst_v.at[r4]], add=True)

        return carry

    lax.fori_loop(0, NCHUNK + 3, _pipe, 0)
    plsc.subcore_barrier()

    def _piece(p, acc):
        pltpu.sync_copy(
            agg_sh.at[pl.ds(sid * ROWS_PT + p * FCH, FCH), :], fbuf)

        @plsc.parallel_loop(0, FCH, 1, unroll=2, carry=acc)
        def _row(r, acc2):
            out = []
            for j in range(HALF // 16):
                x = fbuf[r, pl.ds(16 * j, 16)] + bias_v[pl.ds(16 * j, 16)]
                ex = jnp.exp(2.0 * x)
                th = 1.0 - 2.0 / (ex + 1.0)
                out.append(acc2[j] + th)
            return tuple(out)

        return _row

    acc0 = tuple(jnp.zeros((16,), jnp.float32) for _ in range(HALF // 16))
    acc = lax.fori_loop(0, NF, _piece, acc0)
    total = acc[0]
    for j in range(1, HALF // 16):
        total = total + acc[j]
    accst[...] = total * (1.0 / (N * C))
    pltpu.sync_copy(accst, out_hbm.at[wid])


def kernel(tokens, edge_index, eta, emb, W, bias):
    src = edge_index[0]
    dst = edge_index[1]
    eta4 = jnp.pad(jnp.pad(eta.reshape(E, 3), ((0, 0), (0, 1)))
                   .reshape(E * 4), (0, 16))
    Wp = (W.reshape(F, 3, NCORE, HALF)
          .transpose(2, 0, 1, 3)
          .reshape(NCORE, F, 3 * HALF))
    bias2 = bias.reshape(NCORE, HALF)
    table = _make_table(emb, Wp)
    partials = _sc_edge(src, dst, eta4, tokens, table, bias2)
    return jnp.sum(partials)


# speedup vs baseline: 35.7301x; 1.0003x over previous
import functools

import jax
import jax.numpy as jnp
from jax import lax
from jax.experimental import pallas as pl
from jax.experimental.pallas import tpu as pltpu
from jax.experimental.pallas import tpu_sc as plsc

N = 10000
E = 160000
V = 1000
F = 256
C = 256
HALF = C // 2
NCORE = 2
NSUB = 16
EPW = E // NSUB
B = 40
NCHUNK = EPW // B
ROWS_PT = N // NSUB
FCH = 25
NF = ROWS_PT // FCH
ETAW = B * 4 + 16


def _mm_body(emb_ref, wp_ref, out_ref):
    out_ref[0] = jnp.dot(emb_ref[...], wp_ref[0],
                         preferred_element_type=jnp.float32)


def _make_table(emb, Wp):
    return pl.pallas_call(
        _mm_body,
        grid=(NCORE,),
        in_specs=[
            pl.BlockSpec((V, F), lambda s: (0, 0)),
            pl.BlockSpec((1, F, 3 * HALF), lambda s: (s, 0, 0)),
        ],
        out_specs=pl.BlockSpec((1, V, 3 * HALF), lambda s: (s, 0, 0)),
        out_shape=jax.ShapeDtypeStruct((NCORE, V, 3 * HALF), jnp.float32),
    )(emb, Wp)


_sc_mesh = plsc.VectorSubcoreMesh(core_axis_name="c", subcore_axis_name="s")


@functools.partial(
    pl.kernel,
    out_type=jax.ShapeDtypeStruct((NCORE * NSUB, 16), jnp.float32),
    mesh=_sc_mesh,
    scratch_types=[
        pltpu.VMEM((2, B), jnp.int32),
        pltpu.VMEM((4, B), jnp.int32),
        pltpu.VMEM((2, B), jnp.int32),
        pltpu.VMEM((4 * ETAW,), jnp.float32),
        pltpu.VMEM((2, B, 3 * HALF), jnp.float32),
        pltpu.VMEM((B, HALF), jnp.float32),
        pltpu.VMEM((FCH, HALF), jnp.float32),
        pltpu.VMEM((HALF,), jnp.float32),
        pltpu.VMEM((16,), jnp.float32),
        pltpu.VMEM_SHARED((N, HALF), jnp.float32),
        pltpu.SemaphoreType.DMA,
        pltpu.SemaphoreType.DMA,
        pltpu.SemaphoreType.DMA,
    ],
)
def _sc_edge(src_hbm, dst_hbm, eta_hbm, tok_hbm, t_hbm, bias_hbm, out_hbm,
             src_v, dst_v, tp_v, eta_v, rows_v, msgs_v, fbuf, bias_v,
             accst, agg_sh, sem_a, sem_b, sem_c):
    cid = lax.axis_index("c")
    sid = lax.axis_index("s")
    wid = cid * NSUB + sid

    pltpu.sync_copy(bias_hbm.at[cid], bias_v)

    zero16 = jnp.zeros((16,), jnp.float32)

    def _zrow(r, carry):
        for j in range(HALF // 16):
            fbuf[r, pl.ds(16 * j, 16)] = zero16
        return carry

    lax.fori_loop(0, FCH, _zrow, 0)
    for p in range(NF):
        pltpu.sync_copy(
            fbuf, agg_sh.at[pl.ds(sid * ROWS_PT + p * FCH, FCH), :])
    plsc.subcore_barrier()

    ebase = sid * EPW

    def _a_copies(ch):
        base = ebase + ch * B
        return (
            (src_hbm.at[pl.ds(base, B)], src_v.at[ch % 2]),
            (dst_hbm.at[pl.ds(base, B)], dst_v.at[ch % 4]),
            (eta_hbm.at[pl.ds(base * 4, ETAW)],
             eta_v.at[pl.ds((ch % 4) * ETAW, ETAW)]),
        )

    def _b_copy(ch):
        return (tok_hbm.at[src_v.at[ch % 2]], tp_v.at[ch % 2])

    def _c_copy(ch):
        return (t_hbm.at[cid].at[tp_v.at[ch % 2]], rows_v.at[ch % 2])

    def _pipe(j, carry):
        @pl.when(j >= 3)
        def _():
            pltpu.make_async_copy(*_c_copy(j - 3), sem_c).wait()

        @pl.when(jnp.logical_and(j >= 2, j <= NCHUNK + 1))
        def _():
            pltpu.make_async_copy(*_b_copy(j - 2), sem_b).wait()
            pltpu.async_copy(*_c_copy(j - 2), sem_c)

        @pl.when(jnp.logical_and(j >= 1, j <= NCHUNK))
        def _():
            for s_d in _a_copies(j - 1):
                pltpu.make_async_copy(*s_d, sem_a).wait()
            pltpu.async_copy(*_b_copy(j - 1), sem_b)

        @pl.when(j <= NCHUNK - 1)
        def _():
            for s_d in _a_copies(j):
                pltpu.async_copy(*s_d, sem_a)

        @pl.when(j >= 3)
        def _():
            ch = j - 3
            r2 = ch % 2
            r4 = ch % 4

            @plsc.parallel_loop(0, B, 1, unroll=8)
            def _edge(b):
                ev = eta_v[pl.ds(r4 * ETAW + 4 * b, 16)]
                e0 = ev[0]
                e1 = ev[1]
                e2 = ev[2]
                for j2 in range(HALF // 16):
                    r0 = rows_v[r2, b, pl.ds(16 * j2, 16)]
                    r1 = rows_v[r2, b, pl.ds(HALF + 16 * j2, 16)]
                    r2v = rows_v[r2, b, pl.ds(2 * HALF + 16 * j2, 16)]
                    msgs_v[b, pl.ds(16 * j2, 16)] = (
                        e0 * r0 + e1 * r1 + e2 * r2v)

            pltpu.sync_copy(msgs_v, agg_sh.at[dst_v.at[r4]], add=True)

        return carry

    lax.fori_loop(0, NCHUNK + 3, _pipe, 0)
    plsc.subcore_barrier()

    def _piece(p, acc):
        pltpu.sync_copy(
            agg_sh.at[pl.ds(sid * ROWS_PT + p * FCH, FCH), :], fbuf)

        @plsc.parallel_loop(0, FCH, 1, unroll=2, carry=acc)
        def _row(r, acc2):
            out = []
            for j in range(HALF // 16):
                x = fbuf[r, pl.ds(16 * j, 16)] + bias_v[pl.ds(16 * j, 16)]
                ex = jnp.exp(2.0 * x)
                th = 1.0 - 2.0 / (ex + 1.0)
                out.append(acc2[j] + th)
            return tuple(out)

        return _row

    acc0 = tuple(jnp.zeros((16,), jnp.float32) for _ in range(HALF // 16))
    acc = lax.fori_loop(0, NF, _piece, acc0)
    total = acc[0]
    for j in range(1, HALF // 16):
        total = total + acc[j]
    accst[...] = total * (1.0 / (N * C))
    pltpu.sync_copy(accst, out_hbm.at[wid])


def kernel(tokens, edge_index, eta, emb, W, bias):
    src = edge_index[0]
    dst = edge_index[1]
    eta4 = jnp.pad(jnp.pad(eta.reshape(E, 3), ((0, 0), (0, 1)))
                   .reshape(E * 4), (0, 16))
    Wp = (W.reshape(F, 3, NCORE, HALF)
          .transpose(2, 0, 1, 3)
          .reshape(NCORE, F, 3 * HALF))
    bias2 = bias.reshape(NCORE, HALF)
    table = _make_table(emb, Wp)
    partials = _sc_edge(src, dst, eta4, tokens, table, bias2)
    return jnp.sum(partials)


# speedup vs baseline: 40.1200x; 1.1229x over previous
import functools

import jax
import jax.numpy as jnp
from jax import lax
from jax.experimental import pallas as pl
from jax.experimental.pallas import tpu as pltpu
from jax.experimental.pallas import tpu_sc as plsc

N = 10000
E = 160000
V = 1000
F = 256
C = 256
HALF = C // 2
NCORE = 2
NSUB = 16
EPW = E // NSUB
B = 40
NCHUNK = EPW // B
ROWS_PT = N // NSUB
FCH = 25
NF = ROWS_PT // FCH
ETAW = B * 3 + 16


def _mm_body(emb_ref, wp_ref, out_ref):
    out_ref[0] = jnp.dot(emb_ref[...], wp_ref[0],
                         preferred_element_type=jnp.float32)


def _make_table(emb, Wp):
    return pl.pallas_call(
        _mm_body,
        grid=(NCORE,),
        in_specs=[
            pl.BlockSpec((V, F), lambda s: (0, 0)),
            pl.BlockSpec((1, F, 3 * HALF), lambda s: (s, 0, 0)),
        ],
        out_specs=pl.BlockSpec((1, V, 3 * HALF), lambda s: (s, 0, 0)),
        out_shape=jax.ShapeDtypeStruct((NCORE, V, 3 * HALF), jnp.float32),
    )(emb, Wp)


_sc_mesh = plsc.VectorSubcoreMesh(core_axis_name="c", subcore_axis_name="s")


@functools.partial(
    pl.kernel,
    out_type=jax.ShapeDtypeStruct((NCORE * NSUB, 16), jnp.float32),
    mesh=_sc_mesh,
    scratch_types=[
        pltpu.VMEM((2, B), jnp.int32),
        pltpu.VMEM((4, B), jnp.int32),
        pltpu.VMEM((2, B), jnp.int32),
        pltpu.VMEM((4 * ETAW,), jnp.float32),
        pltpu.VMEM((2, B, 3 * HALF), jnp.float32),
        pltpu.VMEM((B, HALF), jnp.float32),
        pltpu.VMEM((FCH, HALF), jnp.float32),
        pltpu.VMEM((HALF,), jnp.float32),
        pltpu.VMEM((16,), jnp.float32),
        pltpu.VMEM_SHARED((N, HALF), jnp.float32),
        pltpu.SemaphoreType.DMA,
        pltpu.SemaphoreType.DMA,
        pltpu.SemaphoreType.DMA,
    ],
)
def _sc_edge(src_hbm, dst_hbm, eta_hbm, tok_hbm, t_hbm, bias_hbm, out_hbm,
             src_v, dst_v, tp_v, eta_v, rows_v, msgs_v, fbuf, bias_v,
             accst, agg_sh, sem_a, sem_b, sem_c):
    cid = lax.axis_index("c")
    sid = lax.axis_index("s")
    wid = cid * NSUB + sid

    pltpu.sync_copy(bias_hbm.at[cid], bias_v)

    zero16 = jnp.zeros((16,), jnp.float32)

    def _zrow(r, carry):
        for j in range(HALF // 16):
            fbuf[r, pl.ds(16 * j, 16)] = zero16
        return carry

    lax.fori_loop(0, FCH, _zrow, 0)
    for p in range(NF):
        pltpu.sync_copy(
            fbuf, agg_sh.at[pl.ds(sid * ROWS_PT + p * FCH, FCH), :])
    plsc.subcore_barrier()

    ebase = sid * EPW

    def _a_copies(ch):
        base = ebase + ch * B
        return (
            (src_hbm.at[pl.ds(base, B)], src_v.at[ch % 2]),
            (dst_hbm.at[pl.ds(base, B)], dst_v.at[ch % 4]),
            (eta_hbm.at[pl.ds(base * 3, 3 * B)],
             eta_v.at[pl.ds((ch % 4) * ETAW, 3 * B)]),
        )

    def _b_copy(ch):
        return (tok_hbm.at[src_v.at[ch % 2]], tp_v.at[ch % 2])

    def _c_copy(ch):
        return (t_hbm.at[cid].at[tp_v.at[ch % 2]], rows_v.at[ch % 2])

    def _pipe(j, carry):
        @pl.when(j >= 3)
        def _():
            pltpu.make_async_copy(*_c_copy(j - 3), sem_c).wait()

        @pl.when(jnp.logical_and(j >= 2, j <= NCHUNK + 1))
        def _():
            pltpu.make_async_copy(*_b_copy(j - 2), sem_b).wait()
            pltpu.async_copy(*_c_copy(j - 2), sem_c)

        @pl.when(jnp.logical_and(j >= 1, j <= NCHUNK))
        def _():
            for s_d in _a_copies(j - 1):
                pltpu.make_async_copy(*s_d, sem_a).wait()
            pltpu.async_copy(*_b_copy(j - 1), sem_b)

        @pl.when(j <= NCHUNK - 1)
        def _():
            for s_d in _a_copies(j):
                pltpu.async_copy(*s_d, sem_a)

        @pl.when(j >= 3)
        def _():
            ch = j - 3
            r2 = ch % 2
            r4 = ch % 4

            @plsc.parallel_loop(0, B, 1, unroll=8)
            def _edge(b):
                ev = eta_v[pl.ds(r4 * ETAW + 3 * b, 16)]
                e0 = ev[0]
                e1 = ev[1]
                e2 = ev[2]
                for j2 in range(HALF // 16):
                    r0 = rows_v[r2, b, pl.ds(16 * j2, 16)]
                    r1 = rows_v[r2, b, pl.ds(HALF + 16 * j2, 16)]
                    r2v = rows_v[r2, b, pl.ds(2 * HALF + 16 * j2, 16)]
                    msgs_v[b, pl.ds(16 * j2, 16)] = (
                        e0 * r0 + e1 * r1 + e2 * r2v)

            pltpu.sync_copy(msgs_v, agg_sh.at[dst_v.at[r4]], add=True)

        return carry

    lax.fori_loop(0, NCHUNK + 3, _pipe, 0)
    plsc.subcore_barrier()

    def _piece(p, acc):
        pltpu.sync_copy(
            agg_sh.at[pl.ds(sid * ROWS_PT + p * FCH, FCH), :], fbuf)

        @plsc.parallel_loop(0, FCH, 1, unroll=2, carry=acc)
        def _row(r, acc2):
            out = []
            for j in range(HALF // 16):
                x = fbuf[r, pl.ds(16 * j, 16)] + bias_v[pl.ds(16 * j, 16)]
                ex = jnp.exp(2.0 * x)
                th = 1.0 - 2.0 / (ex + 1.0)
                out.append(acc2[j] + th)
            return tuple(out)

        return _row

    acc0 = tuple(jnp.zeros((16,), jnp.float32) for _ in range(HALF // 16))
    acc = lax.fori_loop(0, NF, _piece, acc0)
    total = acc[0]
    for j in range(1, HALF // 16):
        total = total + acc[j]
    accst[...] = total * (1.0 / (N * C))
    pltpu.sync_copy(accst, out_hbm.at[wid])


def kernel(tokens, edge_index, eta, emb, W, bias):
    eta_flat = eta.reshape(E * 3)
    Wp = (W.reshape(F, 3, NCORE, HALF)
          .transpose(2, 0, 1, 3)
          .reshape(NCORE, F, 3 * HALF))
    bias2 = bias.reshape(NCORE, HALF)
    table = _make_table(emb, Wp)
    partials = _sc_edge(edge_index[0], edge_index[1], eta_flat, tokens,
                        table, bias2)
    return jnp.sum(partials)
